# Initial kernel scaffold; baseline (speedup 1.0000x reference)
import jax
import jax.numpy as jnp
from jax.experimental import pallas as pl


def kernel(features, edge_index, W_gcn, b_gcn, skip_gcn, W_pool, b_pool):
    raise NotImplementedError("write your pallas kernel here")


# speedup vs baseline: 6.5236x; 6.5236x over previous
import functools

import jax
import jax.numpy as jnp
from jax import lax
from jax.experimental import pallas as pl
from jax.experimental.pallas import tpu as pltpu
from jax.experimental.pallas import tpu_sc as plsc

_SELU_SCALE = 1.0507009873554805
_SELU_ALPHA = 1.6732632423543772

_NC = 2
_NS = 16
_CH = 128


def _selu(x):
    return _SELU_SCALE * jnp.where(x > 0, x, _SELU_ALPHA * (jnp.exp(x) - 1.0))


def _matmul(x, w):
    n, d_in = x.shape
    d_out = w.shape[1]
    rb = 1000 if n % 1000 == 0 else 8
    grid = n // rb

    def body(x_ref, w_ref, o_ref):
        o_ref[:] = jnp.dot(x_ref[:], w_ref[:], preferred_element_type=jnp.float32)

    return pl.pallas_call(
        body,
        grid=(grid,),
        in_specs=[
            pl.BlockSpec((rb, d_in), lambda i: (i, 0)),
            pl.BlockSpec((d_in, d_out), lambda i: (0, 0)),
        ],
        out_specs=pl.BlockSpec((rb, d_out), lambda i: (i, 0)),
        out_shape=jax.ShapeDtypeStruct((n, d_out), jnp.float32),
    )(x, w)


def _spmm_sc(h, src_r, dst_r, zeros_blk, acc_rows, n_chunks):
    n, d_h = h.shape
    zr = acc_rows // _NS
    mesh = plsc.VectorSubcoreMesh(
        core_axis_name="c", subcore_axis_name="s",
        num_cores=_NC, num_subcores=_NS)

    @functools.partial(
        pl.kernel,
        out_type=jax.ShapeDtypeStruct((_NC, acc_rows, d_h), jnp.float32),
        mesh=mesh,
        scratch_types=[
            pltpu.VMEM((n_chunks, _CH), jnp.int32),
            pltpu.VMEM((n_chunks, _CH), jnp.int32),
            pltpu.VMEM((_CH, d_h), jnp.float32),
            pltpu.VMEM_SHARED((acc_rows, d_h), jnp.float32),
            pltpu.SemaphoreType.DMA,
        ],
        compiler_params=pltpu.CompilerParams(use_tc_tiling_on_sc=False),
    )
    def spmm(h_hbm, src_hbm, dst_hbm, zeros_hbm, out_hbm,
             src_v, dst_v, rows_v, acc_sh, sem):
        c = lax.axis_index("c")
        s = lax.axis_index("s")
        wid = c * _NS + s
        pltpu.sync_copy(src_hbm.at[wid], src_v)
        pltpu.sync_copy(dst_hbm.at[wid], dst_v)
        pltpu.sync_copy(zeros_hbm, acc_sh.at[pl.ds(s * zr, zr)])
        plsc.subcore_barrier()

        def body(j, carry):
            pltpu.async_copy(h_hbm.at[src_v.at[j]], rows_v, sem).wait()
            pltpu.sync_copy(rows_v, acc_sh.at[dst_v.at[j]], add=True)
            return carry

        lax.fori_loop(0, n_chunks, body, 0)
        plsc.subcore_barrier()
        pltpu.sync_copy(acc_sh.at[pl.ds(s * zr, zr)],
                        out_hbm.at[c, pl.ds(s * zr, zr)])

    return spmm(h, src_r, dst_r, zeros_blk)


def _epilogue(h, parts, skip, bg, wp, bp):
    n, d_h = h.shape
    k = wp.shape[1]
    rb = 1000 if n % 1000 == 0 else 8
    grid = n // rb

    def body(h_ref, p_ref, skip_ref, bg_ref, wp_ref, bp_ref,
             asg_ref, pool_ref, acc_ref):
        i = pl.program_id(0)
        agg = p_ref[0] + p_ref[1]
        h2 = _selu(skip_ref[:] * h_ref[:] + agg + bg_ref[:])
        logits = jnp.dot(h2, wp_ref[:], preferred_element_type=jnp.float32)
        logits = logits + bp_ref[:]
        m = jnp.max(logits, axis=-1, keepdims=True)
        e = jnp.exp(logits - m)
        a = e / jnp.sum(e, axis=-1, keepdims=True)
        asg_ref[:] = a
        @pl.when(i == 0)
        def _():
            acc_ref[:] = jnp.zeros_like(acc_ref)
        acc_ref[:] += lax.dot_general(
            a, h2, (((0,), (0,)), ((), ())), preferred_element_type=jnp.float32)
        @pl.when(i == pl.num_programs(0) - 1)
        def _():
            pool_ref[:] = _selu(acc_ref[:])

    asg, pool = pl.pallas_call(
        body,
        grid=(grid,),
        in_specs=[
            pl.BlockSpec((rb, d_h), lambda i: (i, 0)),
            pl.BlockSpec((_NC, rb, d_h), lambda i: (0, i, 0)),
            pl.BlockSpec((1, d_h), lambda i: (0, 0)),
            pl.BlockSpec((1, d_h), lambda i: (0, 0)),
            pl.BlockSpec((d_h, k), lambda i: (0, 0)),
            pl.BlockSpec((1, k), lambda i: (0, 0)),
        ],
        out_specs=[
            pl.BlockSpec((rb, k), lambda i: (i, 0)),
            pl.BlockSpec((k, d_h), lambda i: (0, 0)),
        ],
        out_shape=[
            jax.ShapeDtypeStruct((n, k), jnp.float32),
            jax.ShapeDtypeStruct((k, d_h), jnp.float32),
        ],
        scratch_shapes=[pltpu.VMEM((k, d_h), jnp.float32)],
    )(h, parts, skip, bg, wp, bp)
    return pool, asg


def kernel(features, edge_index, W_gcn, b_gcn, skip_gcn, W_pool, b_pool):
    n, _ = features.shape
    d_h = W_gcn.shape[1]
    e = edge_index.shape[1]
    nw = _NC * _NS

    h = _matmul(features, W_gcn)

    per_tile = -(-e // nw)
    n_chunks = -(-per_tile // _CH)
    e_pad = nw * n_chunks * _CH
    acc_rows = -(-(n + 1) // (_NS * 8)) * (_NS * 8)
    pad = e_pad - e
    src_r = jnp.concatenate(
        [edge_index[0], jnp.zeros((pad,), jnp.int32)]).reshape(nw, n_chunks, _CH)
    dst_r = jnp.concatenate(
        [edge_index[1], jnp.full((pad,), n, jnp.int32)]).reshape(nw, n_chunks, _CH)
    zeros_blk = jnp.zeros((acc_rows // _NS, d_h), jnp.float32)

    parts = _spmm_sc(h, src_r, dst_r, zeros_blk, acc_rows, n_chunks)

    pool, asg = _epilogue(
        h, parts,
        skip_gcn.reshape(1, d_h), b_gcn.reshape(1, d_h),
        W_pool, b_pool.reshape(1, -1))
    return (pool, asg)
